# baseline (speedup 1.0000x reference)
import functools

import jax
import jax.numpy as jnp
from jax import lax
from jax.experimental import pallas as pl
from jax.experimental.pallas import tpu as pltpu
from jax.experimental.pallas import tpu_sc as plsc

_NUM_WORKERS = 32
_CHUNK = 128
_NBUF = 8
_DIM = 32


@functools.partial(jax.jit, static_argnums=(2, 3))
def _emb_lookup(idx_t, weight, n_rows, n_tok):
    bs_per_w = n_rows // _NUM_WORKERS
    blocks_per_w = bs_per_w // _CHUNK
    n_chunks = n_tok * blocks_per_w
    n_groups = n_chunks // _NBUF
    mesh = plsc.VectorSubcoreMesh(core_axis_name="c", subcore_axis_name="s")

    @functools.partial(
        pl.kernel,
        out_type=jax.ShapeDtypeStruct((n_tok, n_rows, _DIM), jnp.float32),
        mesh=mesh,
        scratch_types=[
            pltpu.VMEM((n_tok, bs_per_w), jnp.int32),
            pltpu.VMEM((2, _NBUF, _CHUNK, _DIM), jnp.float32),
            pltpu.SemaphoreType.DMA((2,)),
            pltpu.SemaphoreType.DMA((2,)),
        ],
        compiler_params=pltpu.CompilerParams(use_tc_tiling_on_sc=False),
    )
    def body(idx_hbm, table_hbm, out_hbm, idx_v, sbuf, gsem, ssem):
        wid = lax.axis_index("s") * 2 + lax.axis_index("c")
        base = wid * bs_per_w
        pltpu.sync_copy(idx_hbm.at[:, pl.ds(base, bs_per_w)], idx_v)

        def gather_desc(j, b, sb):
            c = j * _NBUF + b
            t = lax.rem(c, n_tok)
            blk = c // n_tok
            return pltpu.make_async_copy(
                table_hbm.at[idx_v.at[t, pl.ds(blk * _CHUNK, _CHUNK)]],
                sbuf.at[sb, b],
                gsem.at[sb],
            )

        def scatter_desc(j, b, sb):
            c = j * _NBUF + b
            t = lax.rem(c, n_tok)
            blk = c // n_tok
            return pltpu.make_async_copy(
                sbuf.at[sb, b],
                out_hbm.at[t, pl.ds(base + blk * _CHUNK, _CHUNK)],
                ssem.at[sb],
            )

        def launch_gathers(j, sb):
            for b in range(_NBUF):
                gather_desc(j, b, sb).start()

        def wait_gathers(j, sb):
            for b in range(_NBUF):
                gather_desc(j, b, sb).wait()

        def launch_scatters(j, sb):
            for b in range(_NBUF):
                scatter_desc(j, b, sb).start()

        def wait_scatters(j, sb):
            for b in range(_NBUF):
                scatter_desc(j, b, sb).wait()

        launch_gathers(0, 0)

        def group(g, carry):
            sb = lax.rem(g, 2)
            wait_gathers(g, sb)
            launch_scatters(g, sb)

            @pl.when(g + 1 < n_groups)
            def _():
                @pl.when(g >= 1)
                def _():
                    wait_scatters(g - 1, 1 - sb)

                launch_gathers(g + 1, 1 - sb)

            return carry

        lax.fori_loop(0, n_groups, group, 0)
        wait_scatters(n_groups - 2, n_groups % 2)
        wait_scatters(n_groups - 1, (n_groups - 1) % 2)

    return body(idx_t, weight)


def _tp_body(w0, w1, w2, w3, o_ref):
    o_ref[:, 0:32] = w0[...].T
    o_ref[:, 32:64] = w1[...].T
    o_ref[:, 64:96] = w2[...].T
    o_ref[:, 96:128] = w3[...].T


def _transpose_pack(weight):
    wt = weight.T
    n = wt.shape[1]
    grid = (n + 511) // 512
    return pl.pallas_call(
        _tp_body,
        grid=(grid,),
        in_specs=[
            pl.BlockSpec((_DIM, 128), lambda i, a=a: (0, 4 * i + a))
            for a in range(4)
        ],
        out_specs=pl.BlockSpec((128, 128), lambda i: (i, 0)),
        out_shape=jax.ShapeDtypeStruct((grid * 128, 128), jnp.float32),
    )(wt, wt, wt, wt)


def kernel(token_ids, weight):
    n_rows, n_tok = token_ids.shape
    w128 = _transpose_pack(weight)
    w_lin = w128.reshape(w128.shape[0] * 4, _DIM)
    r = jnp.maximum(token_ids.astype(jnp.int32), 0)
    idx_t = (((r >> 9) << 9) | ((r & 127) << 2) | ((r >> 7) & 3)).T
    out_t = _emb_lookup(idx_t, w_lin, n_rows, n_tok)
    return out_t.transpose(1, 0, 2)


# speedup vs baseline: 1.9626x; 1.9626x over previous
import functools

import jax
import jax.numpy as jnp
from jax import lax
from jax.experimental import pallas as pl
from jax.experimental.pallas import tpu as pltpu
from jax.experimental.pallas import tpu_sc as plsc

_NUM_WORKERS = 32
_CHUNK = 128
_NBUF = 8
_DIM = 32


@functools.partial(jax.jit, static_argnums=(2, 3))
def _emb_lookup(idx_t, weight, n_rows, n_tok):
    bs_per_w = n_rows // _NUM_WORKERS
    blocks_per_w = bs_per_w // _CHUNK
    n_chunks = n_tok * blocks_per_w
    n_groups = n_chunks // _NBUF
    mesh = plsc.VectorSubcoreMesh(core_axis_name="c", subcore_axis_name="s")

    @functools.partial(
        pl.kernel,
        out_type=jax.ShapeDtypeStruct((n_tok, n_rows, _DIM), jnp.float32),
        mesh=mesh,
        scratch_types=[
            pltpu.VMEM((n_tok, bs_per_w), jnp.int32),
            pltpu.VMEM((2, _NBUF, _CHUNK, _DIM), jnp.float32),
            pltpu.SemaphoreType.DMA((2,)),
            pltpu.SemaphoreType.DMA((2,)),
        ],
        compiler_params=pltpu.CompilerParams(use_tc_tiling_on_sc=False),
    )
    def body(idx_hbm, table_hbm, out_hbm, idx_v, sbuf, gsem, ssem):
        wid = lax.axis_index("s") * 2 + lax.axis_index("c")
        base = wid * bs_per_w
        pltpu.sync_copy(idx_hbm.at[:, pl.ds(base, bs_per_w)], idx_v)

        def gather_desc(j, b, sb):
            c = j * _NBUF + b
            t = lax.rem(c, n_tok)
            blk = c // n_tok
            return pltpu.make_async_copy(
                table_hbm.at[idx_v.at[t, pl.ds(blk * _CHUNK, _CHUNK)]],
                sbuf.at[sb, b],
                gsem.at[sb],
            )

        def scatter_desc(j, b, sb):
            c = j * _NBUF + b
            t = lax.rem(c, n_tok)
            blk = c // n_tok
            return pltpu.make_async_copy(
                sbuf.at[sb, b],
                out_hbm.at[t, pl.ds(base + blk * _CHUNK, _CHUNK)],
                ssem.at[sb],
            )

        def launch_gathers(j, sb):
            for b in range(_NBUF):
                gather_desc(j, b, sb).start()

        def wait_gathers(j, sb):
            for b in range(_NBUF):
                gather_desc(j, b, sb).wait()

        def launch_scatters(j, sb):
            for b in range(_NBUF):
                scatter_desc(j, b, sb).start()

        def wait_scatters(j, sb):
            for b in range(_NBUF):
                scatter_desc(j, b, sb).wait()

        launch_gathers(0, 0)

        def group(g, carry):
            sb = lax.rem(g, 2)
            wait_gathers(g, sb)
            launch_scatters(g, sb)

            @pl.when(g + 1 < n_groups)
            def _():
                @pl.when(g >= 1)
                def _():
                    wait_scatters(g - 1, 1 - sb)

                launch_gathers(g + 1, 1 - sb)

            return carry

        lax.fori_loop(0, n_groups, group, 0)
        wait_scatters(n_groups - 2, n_groups % 2)
        wait_scatters(n_groups - 1, (n_groups - 1) % 2)

    return body(idx_t, weight)


def kernel(token_ids, weight):
    n_rows, n_tok = token_ids.shape
    idx_t = jnp.maximum(token_ids.astype(jnp.int32), 0).T
    out_t = _emb_lookup(idx_t, weight, n_rows, n_tok)
    return out_t.transpose(1, 0, 2)
